# baseline (speedup 1.0000x reference)
import jax
import jax.numpy as jnp
from jax.experimental import pallas as pl
from jax.experimental.pallas import tpu as pltpu
from jax.experimental.pallas import tpu_sc as plsc

_ROWS = 16
_LANES = 16


def kernel(x, pos_table):
    B, S, D = x.shape
    pos = pos_table[:S]
    xf = x.reshape(B * S, D)
    n_seq_blocks = S // _ROWS

    mesh = plsc.VectorSubcoreMesh(core_axis_name="core",
                                  subcore_axis_name="subcore")

    @pl.kernel(out_type=jax.ShapeDtypeStruct((B * S, D), x.dtype), mesh=mesh)
    def sc_kernel(x_hbm, pos_hbm, o_hbm):
        def body(x_vmem, pos_vmem, o_vmem):
            @pl.loop(0, _ROWS)
            def _(r):
                @pl.loop(0, D, step=_LANES)
                def _(c):
                    slc = (pl.ds(r, 1), pl.ds(c, _LANES))
                    o_vmem.at[*slc][...] = (
                        x_vmem.at[*slc][...] + pos_vmem.at[*slc][...]
                    )

        pltpu.emit_pipeline(
            body,
            grid=(n_seq_blocks, B),
            in_specs=[
                pl.BlockSpec((_ROWS, D), index_map=lambda s, b: (b * n_seq_blocks + s, 0)),
                pl.BlockSpec((_ROWS, D), index_map=lambda s, b: (s, 0)),
            ],
            out_specs=[
                pl.BlockSpec((_ROWS, D), index_map=lambda s, b: (b * n_seq_blocks + s, 0)),
            ],
            core_axis_name=("core", "subcore"),
            dimension_semantics=(pltpu.PARALLEL, pltpu.ARBITRARY),
        )(x_hbm, pos_hbm, o_hbm)

    return sc_kernel(xf, pos).reshape(B, S, D)


# speedup vs baseline: 4.5425x; 4.5425x over previous
import jax
import jax.numpy as jnp
from jax.experimental import pallas as pl
from jax.experimental.pallas import tpu as pltpu

_CHUNK = 512
_K = 4


def _make_body(n_chunks, n_pos_chunks):
    def body(x_hbm, pos_hbm, o_hbm, xbuf, pbuf, obuf, xsem, psem, osem):
        def start_x(i):
            slot = i % _K
            pltpu.make_async_copy(
                x_hbm.at[pl.ds(i * _CHUNK, _CHUNK), :],
                xbuf.at[slot], xsem.at[slot]).start()

        def start_pos(j):
            pltpu.make_async_copy(
                pos_hbm.at[pl.ds(j * _CHUNK, _CHUNK), :],
                pbuf.at[j], psem.at[j]).start()

        for j in range(min(n_pos_chunks, _K)):
            start_pos(j)
        for i in range(_K - 1):
            start_x(i)

        for i in range(n_chunks):
            slot = i % _K
            pslot = i % n_pos_chunks
            if i + _K - 1 < n_chunks:
                start_x(i + _K - 1)
            if _K <= i + _K < n_pos_chunks:
                start_pos(i + _K)
            pltpu.make_async_copy(
                x_hbm.at[pl.ds(i * _CHUNK, _CHUNK), :],
                xbuf.at[slot], xsem.at[slot]).wait()
            if i < n_pos_chunks:
                pltpu.make_async_copy(
                    pos_hbm.at[pl.ds(i * _CHUNK, _CHUNK), :],
                    pbuf.at[i], psem.at[i]).wait()
            if i >= _K:
                pltpu.make_async_copy(
                    obuf.at[slot],
                    o_hbm.at[pl.ds((i - _K) * _CHUNK, _CHUNK), :],
                    osem.at[slot]).wait()
            obuf[slot, :, :] = xbuf[slot, :, :] + pbuf[pslot, :, :]
            pltpu.make_async_copy(
                obuf.at[slot],
                o_hbm.at[pl.ds(i * _CHUNK, _CHUNK), :],
                osem.at[slot]).start()

        for i in range(max(n_chunks - _K, 0), n_chunks):
            slot = i % _K
            pltpu.make_async_copy(
                obuf.at[slot],
                o_hbm.at[pl.ds(i * _CHUNK, _CHUNK), :],
                osem.at[slot]).wait()

    return body


def kernel(x, pos_table):
    B, S, D = x.shape
    pos = pos_table[:S]
    xf = x.reshape(B * S, D)
    n_chunks = (B * S) // _CHUNK
    n_pos_chunks = S // _CHUNK
    out = pl.pallas_call(
        _make_body(n_chunks, n_pos_chunks),
        in_specs=[
            pl.BlockSpec(memory_space=pltpu.MemorySpace.HBM),
            pl.BlockSpec(memory_space=pltpu.MemorySpace.HBM),
        ],
        out_specs=pl.BlockSpec(memory_space=pltpu.MemorySpace.HBM),
        out_shape=jax.ShapeDtypeStruct((B * S, D), x.dtype),
        scratch_shapes=[
            pltpu.VMEM((_K, _CHUNK, D), x.dtype),
            pltpu.VMEM((n_pos_chunks, _CHUNK, D), x.dtype),
            pltpu.VMEM((_K, _CHUNK, D), x.dtype),
            pltpu.SemaphoreType.DMA((_K,)),
            pltpu.SemaphoreType.DMA((n_pos_chunks,)),
            pltpu.SemaphoreType.DMA((_K,)),
        ],
    )(xf, pos)
    return out.reshape(B, S, D)


# speedup vs baseline: 4.5758x; 1.0073x over previous
import jax
import jax.numpy as jnp
from jax.experimental import pallas as pl
from jax.experimental.pallas import tpu as pltpu

_CHUNK = 1024
_K = 4


def _make_body(n_chunks, n_pos_chunks):
    def body(x_hbm, pos_hbm, o_hbm, xbuf, pbuf, obuf, xsem, psem, osem):
        def start_x(i):
            slot = i % _K
            pltpu.make_async_copy(
                x_hbm.at[pl.ds(i * _CHUNK, _CHUNK), :],
                xbuf.at[slot], xsem.at[slot]).start()

        def start_pos(j):
            pltpu.make_async_copy(
                pos_hbm.at[pl.ds(j * _CHUNK, _CHUNK), :],
                pbuf.at[j], psem.at[j]).start()

        for j in range(min(n_pos_chunks, _K)):
            start_pos(j)
        for i in range(_K - 1):
            start_x(i)

        for i in range(n_chunks):
            slot = i % _K
            pslot = i % n_pos_chunks
            if i + _K - 1 < n_chunks:
                start_x(i + _K - 1)
            if _K <= i + _K < n_pos_chunks:
                start_pos(i + _K)
            pltpu.make_async_copy(
                x_hbm.at[pl.ds(i * _CHUNK, _CHUNK), :],
                xbuf.at[slot], xsem.at[slot]).wait()
            if i < n_pos_chunks:
                pltpu.make_async_copy(
                    pos_hbm.at[pl.ds(i * _CHUNK, _CHUNK), :],
                    pbuf.at[i], psem.at[i]).wait()
            if i >= _K:
                pltpu.make_async_copy(
                    obuf.at[slot],
                    o_hbm.at[pl.ds((i - _K) * _CHUNK, _CHUNK), :],
                    osem.at[slot]).wait()
            obuf[slot, :, :] = xbuf[slot, :, :] + pbuf[pslot, :, :]
            pltpu.make_async_copy(
                obuf.at[slot],
                o_hbm.at[pl.ds(i * _CHUNK, _CHUNK), :],
                osem.at[slot]).start()

        for i in range(max(n_chunks - _K, 0), n_chunks):
            slot = i % _K
            pltpu.make_async_copy(
                obuf.at[slot],
                o_hbm.at[pl.ds(i * _CHUNK, _CHUNK), :],
                osem.at[slot]).wait()

    return body


def kernel(x, pos_table):
    B, S, D = x.shape
    pos = pos_table[:S]
    xf = x.reshape(B * S, D)
    n_chunks = (B * S) // _CHUNK
    n_pos_chunks = S // _CHUNK
    out = pl.pallas_call(
        _make_body(n_chunks, n_pos_chunks),
        in_specs=[
            pl.BlockSpec(memory_space=pltpu.MemorySpace.HBM),
            pl.BlockSpec(memory_space=pltpu.MemorySpace.HBM),
        ],
        out_specs=pl.BlockSpec(memory_space=pltpu.MemorySpace.HBM),
        out_shape=jax.ShapeDtypeStruct((B * S, D), x.dtype),
        scratch_shapes=[
            pltpu.VMEM((_K, _CHUNK, D), x.dtype),
            pltpu.VMEM((n_pos_chunks, _CHUNK, D), x.dtype),
            pltpu.VMEM((_K, _CHUNK, D), x.dtype),
            pltpu.SemaphoreType.DMA((_K,)),
            pltpu.SemaphoreType.DMA((n_pos_chunks,)),
            pltpu.SemaphoreType.DMA((_K,)),
        ],
    )(xf, pos)
    return out.reshape(B, S, D)
